# baseline (speedup 1.0000x reference)
import functools

import jax
import jax.numpy as jnp
from jax import lax
from jax.experimental import pallas as pl
from jax.experimental.pallas import tpu as pltpu
from jax.experimental.pallas import tpu_sc as plsc

B = 16384 * 50
D = 64
P = 8
W = P * D
NC = 2
NS = 16
NW = NC * NS
BPW = B // NW
CHUNK = 512
NO = CHUNK // P
NCHUNKS = BPW // CHUNK
OPW = BPW // P
NVEC = NO // 16

def _quad_code(x):
    a = x & 0xFF
    b = lax.shift_right_logical(x, 8) & 0xFF
    c = lax.shift_right_logical(x, 16) & 0xFF
    d = lax.shift_right_logical(x, 24)
    return a * 27 + b * 9 + c * 3 + d


def _make_kernel(ncores, ogroups):
    mesh = plsc.VectorSubcoreMesh(
        core_axis_name="c", subcore_axis_name="s", num_cores=ncores
    )
    opw = ogroups // (ncores * NS)
    nchunks = opw // NO

    @functools.partial(
        pl.kernel,
        mesh=mesh,
        out_type=jax.ShapeDtypeStruct((ogroups, W), jnp.float32),
        scratch_types=[
            pltpu.VMEM((2, NO), jnp.int32),
            pltpu.VMEM((2, NO), jnp.int32),
            pltpu.VMEM((2, NO), jnp.int32),
            pltpu.VMEM((2, NO, W), jnp.float32),
            pltpu.SemaphoreType.DMA,
            pltpu.SemaphoreType.DMA,
            pltpu.SemaphoreType.DMA,
            pltpu.SemaphoreType.DMA,
            pltpu.SemaphoreType.DMA,
            pltpu.SemaphoreType.DMA,
        ],
    )
    def _emb_lookup(ev_hbm, od_hbm, otable_hbm, out_hbm,
                    ev_v, od_v, oct_v, rows_v,
                    si0, si1, sg0, sg1, sw0, sw1):
        wid = lax.axis_index("s") * ncores + lax.axis_index("c")
        obase = wid * opw

        s_idx = (si0, si1)
        s_wr = (sw0, sw1)
        s_g = (sg0, sg1)

        def idx_start(c, b):
            off = pl.multiple_of(obase + c * NO, NO)
            pltpu.async_copy(ev_hbm.at[pl.ds(off, NO)], ev_v.at[b], s_idx[b])
            pltpu.async_copy(od_hbm.at[pl.ds(off, NO)], od_v.at[b], s_idx[b])

        def idx_wait(b):
            pltpu.make_async_copy(
                ev_hbm.at[pl.ds(0, NO)], ev_v.at[b], s_idx[b]
            ).wait()
            pltpu.make_async_copy(
                od_hbm.at[pl.ds(0, NO)], od_v.at[b], s_idx[b]
            ).wait()

        def write_wait(b):
            pltpu.make_async_copy(
                rows_v.at[b], out_hbm.at[pl.ds(0, NO), :], s_wr[b]
            ).wait()

        def do_chunk(i, c, b):
            idx_wait(b)
            for v in range(NVEC):
                e = _quad_code(ev_v[b, pl.ds(v * 16, 16)])
                o = _quad_code(od_v[b, pl.ds(v * 16, 16)])
                oct_v[b, pl.ds(v * 16, 16)] = e * 81 + o

            @pl.when(i > 0)
            def _():
                write_wait(b)

            gather = pltpu.async_copy(
                otable_hbm.at[oct_v.at[b]], rows_v.at[b], s_g[b]
            )

            @pl.when(c + 1 < nchunks)
            def _():
                idx_start(c + 1, b ^ 1)

            gather.wait()
            off = pl.multiple_of(obase + c * NO, NO)
            pltpu.async_copy(
                rows_v.at[b], out_hbm.at[pl.ds(off, NO), :], s_wr[b]
            )

        idx_start(0, 0)

        def body(i, carry):
            do_chunk(i, 2 * i, 0)
            do_chunk(i, 2 * i + 1, 1)
            return carry

        lax.fori_loop(0, nchunks // 2, body, 0)

        write_wait(0)
        write_wait(1)

    return _emb_lookup


_emb_full = _make_kernel(NC, B // P)
_emb_half = _make_kernel(1, B // P // 2)


def kernel(indices, table):
    words = lax.bitcast_convert_type(
        indices.astype(jnp.int8).reshape(B // 4, 4), jnp.int32
    )
    ev = lax.slice(words, (0,), (B // 4,), (2,))
    od = lax.slice(words, (1,), (B // 4,), (2,))
    codes = jnp.arange(81)
    qtable = jnp.concatenate(
        [
            table[codes // 27],
            table[(codes // 9) % 3],
            table[(codes // 3) % 3],
            table[codes % 3],
        ],
        axis=1,
    )
    codes8 = jnp.arange(6561)
    otable = jnp.concatenate([qtable[codes8 // 81], qtable[codes8 % 81]], axis=1)
    h = B // 8
    o0 = _emb_half(ev[: h // 2], od[: h // 2], otable)
    o1 = _emb_half(ev[h // 2:], od[h // 2:], otable)
    out = jnp.concatenate([o0, o1], axis=0)
    return out.reshape(indices.shape[0], indices.shape[1], D)


# speedup vs baseline: 1.3070x; 1.3070x over previous
import functools

import jax
import jax.numpy as jnp
from jax import lax
from jax.experimental import pallas as pl
from jax.experimental.pallas import tpu as pltpu
from jax.experimental.pallas import tpu_sc as plsc

B = 16384 * 50
D = 64
P = 8
W = P * D
NC = 2
NS = 16
NW = NC * NS
BPW = B // NW
CHUNK = 640
NO = CHUNK // P
NCHUNKS = BPW // CHUNK
OPW = BPW // P
NVEC = NO // 16

def _quad_code(x):
    a = x & 0xFF
    b = lax.shift_right_logical(x, 8) & 0xFF
    c = lax.shift_right_logical(x, 16) & 0xFF
    d = lax.shift_right_logical(x, 24)
    return a * 27 + b * 9 + c * 3 + d


def _make_kernel(ncores, ogroups):
    mesh = plsc.VectorSubcoreMesh(
        core_axis_name="c", subcore_axis_name="s", num_cores=ncores
    )
    opw = ogroups // (ncores * NS)
    nchunks = opw // NO

    @functools.partial(
        pl.kernel,
        mesh=mesh,
        out_type=jax.ShapeDtypeStruct((ogroups, W), jnp.float32),
        scratch_types=[
            pltpu.VMEM((2, NO), jnp.int32),
            pltpu.VMEM((2, NO), jnp.int32),
            pltpu.VMEM((2, NO), jnp.int32),
            pltpu.VMEM((2, NO, W), jnp.float32),
            pltpu.SemaphoreType.DMA,
            pltpu.SemaphoreType.DMA,
            pltpu.SemaphoreType.DMA,
            pltpu.SemaphoreType.DMA,
            pltpu.SemaphoreType.DMA,
            pltpu.SemaphoreType.DMA,
        ],
    )
    def _emb_lookup(ev_hbm, od_hbm, otable_hbm, out_hbm,
                    ev_v, od_v, oct_v, rows_v,
                    si0, si1, sg0, sg1, sw0, sw1):
        wid = lax.axis_index("s") * ncores + lax.axis_index("c")
        obase = wid * opw

        s_idx = (si0, si1)
        s_wr = (sw0, sw1)
        s_g = (sg0, sg1)

        def idx_start(c, b):
            off = pl.multiple_of(obase + c * NO, NO)
            pltpu.async_copy(ev_hbm.at[pl.ds(off, NO)], ev_v.at[b], s_idx[b])
            pltpu.async_copy(od_hbm.at[pl.ds(off, NO)], od_v.at[b], s_idx[b])

        def idx_wait(b):
            pltpu.make_async_copy(
                ev_hbm.at[pl.ds(0, NO)], ev_v.at[b], s_idx[b]
            ).wait()
            pltpu.make_async_copy(
                od_hbm.at[pl.ds(0, NO)], od_v.at[b], s_idx[b]
            ).wait()

        def write_wait(b):
            pltpu.make_async_copy(
                rows_v.at[b], out_hbm.at[pl.ds(0, NO), :], s_wr[b]
            ).wait()

        def do_chunk(i, c, b):
            idx_wait(b)
            for v in range(NVEC):
                e = _quad_code(ev_v[b, pl.ds(v * 16, 16)])
                o = _quad_code(od_v[b, pl.ds(v * 16, 16)])
                oct_v[b, pl.ds(v * 16, 16)] = e * 81 + o

            @pl.when(i > 0)
            def _():
                write_wait(b)

            gather = pltpu.async_copy(
                otable_hbm.at[oct_v.at[b]], rows_v.at[b], s_g[b]
            )

            @pl.when(c + 1 < nchunks)
            def _():
                idx_start(c + 1, b ^ 1)

            gather.wait()
            off = pl.multiple_of(obase + c * NO, NO)
            pltpu.async_copy(
                rows_v.at[b], out_hbm.at[pl.ds(off, NO), :], s_wr[b]
            )

        idx_start(0, 0)

        def body(i, carry):
            do_chunk(i, 2 * i, 0)
            do_chunk(i, 2 * i + 1, 1)
            return carry

        lax.fori_loop(0, nchunks // 2, body, 0)

        write_wait(0)
        write_wait(1)

    return _emb_lookup


_emb_full = _make_kernel(NC, B // P)


def kernel(indices, table):
    words = lax.bitcast_convert_type(
        indices.astype(jnp.int8).reshape(B // 4, 4), jnp.int32
    )
    ev = lax.slice(words, (0,), (B // 4,), (2,))
    od = lax.slice(words, (1,), (B // 4,), (2,))
    codes = jnp.arange(81)
    qtable = jnp.concatenate(
        [
            table[codes // 27],
            table[(codes // 9) % 3],
            table[(codes // 3) % 3],
            table[codes % 3],
        ],
        axis=1,
    )
    codes8 = jnp.arange(6561)
    otable = jnp.concatenate([qtable[codes8 // 81], qtable[codes8 % 81]], axis=1)
    out = _emb_full(ev, od, otable)
    return out.reshape(indices.shape[0], indices.shape[1], D)
